# baseline (speedup 1.0000x reference)
import functools
import math

import jax
import jax.numpy as jnp
from jax import lax
from jax.experimental import pallas as pl
from jax.experimental.pallas import tpu as pltpu
from jax.experimental.pallas import tpu_sc as plsc

NC = 2
NS = 16
L = 16
CH = 128


def _round_up(a, b):
    return (a + b - 1) // b * b


SK = 4


@functools.lru_cache(maxsize=None)
def _sc_segment_sum(npad, cpt, feat, with_count):
    fh = feat // NC
    rows_per_tile = npad // NS
    nz_full = rows_per_tile // CH
    z_rem = rows_per_tile - nz_full * CH
    mesh = plsc.VectorSubcoreMesh(core_axis_name="c", subcore_axis_name="s")

    out_type = [jax.ShapeDtypeStruct((NC, npad, fh), jnp.float32)]
    if with_count:
        out_type.append(jax.ShapeDtypeStruct((NS * npad,), jnp.float32))

    @functools.partial(
        pl.kernel,
        mesh=mesh,
        out_type=out_type,
        compiler_params=pltpu.CompilerParams(
            needs_layout_passes=False, use_tc_tiling_on_sc=False),
        scratch_types=[
            pltpu.VMEM((SK * CH,), jnp.int32),
            pltpu.VMEM((SK * CH,), jnp.int32),
            pltpu.VMEM((SK * CH,), jnp.int32),
            pltpu.VMEM((SK * CH,), jnp.int32),
            pltpu.VMEM((SK * CH, fh), jnp.float32),
            pltpu.VMEM((SK * CH, fh), jnp.float32),
            pltpu.VMEM((npad,), jnp.float32),
            pltpu.VMEM_SHARED((npad, fh), jnp.float32),
            pltpu.SemaphoreType.DMA,
            pltpu.SemaphoreType.DMA,
            pltpu.SemaphoreType.DMA,
            pltpu.SemaphoreType.DMA,
            pltpu.SemaphoreType.DMA,
            pltpu.SemaphoreType.DMA,
        ],
    )
    def seg(t_hbm, src_hbm, dst_hbm, *out_and_scratch):
        if with_count:
            out_hbm, cnt_hbm = out_and_scratch[:2]
            rest = out_and_scratch[2:]
        else:
            out_hbm = out_and_scratch[0]
            rest = out_and_scratch[1:]
        (s0, s1, d0, d1, rows0, rows1, cnt, acc,
         g0, g1, i0, i1, x0, x1) = rest
        sidxb = (s0, s1)
        didxb = (d0, d1)
        rows = (rows0, rows1)
        gsem = (g0, g1)
        isem = (i0, i1)
        ssem = (x0, x1)
        zbuf = rows0.at[pl.ds(0, CH)]
        c = lax.axis_index("c")
        s = lax.axis_index("s")
        row0 = s * rows_per_tile

        zv = jnp.zeros((L,), dtype=jnp.float32)

        def zrow(i, carry):
            for g in range(fh // L):
                zbuf[i, pl.ds(g * L, L)] = zv
            return carry

        lax.fori_loop(0, CH, zrow, 0)
        for z in range(nz_full):
            pltpu.sync_copy(zbuf, acc.at[pl.ds(row0 + z * CH, CH)])
        if z_rem:
            pltpu.sync_copy(zbuf.at[pl.ds(0, z_rem)],
                            acc.at[pl.ds(row0 + nz_full * CH, z_rem)])
        if with_count:
            def zi(i, carry):
                cnt[pl.ds(i * L, L)] = zv
                return carry

            lax.fori_loop(0, npad // L, zi, 0)
        plsc.subcore_barrier()

        ones = jnp.full((L,), 1.0, dtype=jnp.float32)
        nsuper = cpt // SK
        jrow0 = s * nsuper

        pltpu.sync_copy(src_hbm.at[jrow0], sidxb[0])
        pltpu.sync_copy(dst_hbm.at[jrow0], didxb[0])
        pltpu.async_copy(src_hbm.at[jrow0 + 1], sidxb[1], isem[1])
        pltpu.async_copy(dst_hbm.at[jrow0 + 1], didxb[1], isem[1])
        HB = SK * CH // 2
        pltpu.async_copy(t_hbm.at[c].at[sidxb[0].at[pl.ds(0, HB)]],
                         rows[0].at[pl.ds(0, HB)], gsem[0])
        pltpu.async_copy(t_hbm.at[c].at[sidxb[0].at[pl.ds(HB, HB)]],
                         rows[0].at[pl.ds(HB, HB)], gsem[0])

        def pair(p, carry):
            for b in range(2):
                j = 2 * p + b
                for hh in range(2):
                    pltpu.make_async_copy(
                        t_hbm.at[c].at[sidxb[b].at[pl.ds(hh * HB, HB)]],
                        rows[b].at[pl.ds(hh * HB, HB)], gsem[b]).wait()
                scat = pltpu.async_copy(
                    rows[b], acc.at[didxb[b]], ssem[b], add=True)
                nxt = j + 1

                @pl.when(nxt < nsuper)
                def _():
                    pltpu.make_async_copy(
                        src_hbm.at[jrow0 + nxt], sidxb[1 - b],
                        isem[1 - b]).wait()
                    pltpu.make_async_copy(
                        dst_hbm.at[jrow0 + nxt], didxb[1 - b],
                        isem[1 - b]).wait()
                    for hh in range(2):
                        pltpu.async_copy(
                            t_hbm.at[c].at[sidxb[1 - b].at[pl.ds(hh * HB,
                                                                 HB)]],
                            rows[1 - b].at[pl.ds(hh * HB, HB)],
                            gsem[1 - b])

                if with_count:
                    @pl.when(c == 0)
                    def _():
                        for i in range(SK * CH // L):
                            d = didxb[b][pl.ds(i * L, L)]
                            plsc.addupdate_scatter(cnt, (d,), ones)
                scat.wait()

                @pl.when(j + 2 < nsuper)
                def _():
                    pltpu.async_copy(src_hbm.at[jrow0 + j + 2],
                                     sidxb[b], isem[b])
                    pltpu.async_copy(dst_hbm.at[jrow0 + j + 2],
                                     didxb[b], isem[b])
            return carry

        lax.fori_loop(0, nsuper // 2, pair, 0)

        if with_count:
            @pl.when(c == 0)
            def _():
                pltpu.sync_copy(cnt, cnt_hbm.at[pl.ds(s * npad, npad)])
        plsc.subcore_barrier()

        pltpu.sync_copy(acc.at[pl.ds(row0, rows_per_tile)],
                        out_hbm.at[c].at[pl.ds(row0, rows_per_tile)])

    return seg


def _tc_matmul(xp, w, blk):
    npad, d = xp.shape
    h = w.shape[1]

    def body(xb, wb, ob):
        ob[...] = jnp.dot(xb[...], wb[...], preferred_element_type=jnp.float32)

    return pl.pallas_call(
        body,
        grid=(npad // blk,),
        in_specs=[
            pl.BlockSpec((blk, d), lambda i: (i, 0)),
            pl.BlockSpec((d, h), lambda i: (0, 0)),
        ],
        out_specs=pl.BlockSpec((blk, h), lambda i: (i, 0)),
        out_shape=jax.ShapeDtypeStruct((npad, h), jnp.float32),
    )(xp, w)


def _tc_combine(pa, cntf, xp, wr, wnext, b, blk):
    npad, d = xp.shape
    h = wr.shape[1]
    fh = h // NC

    def body(pb, cb, xb, wrb, wnb, bb, hb, tb):
        cw = cb[...]
        inv = 1.0 / jnp.maximum(jnp.sum(cw, axis=0), 1.0)
        p = pb[...]
        agg = jnp.concatenate([p[i] for i in range(NC)], axis=-1)
        hv = jnp.maximum(
            agg * inv[:, None] + bb[...]
            + jnp.dot(xb[...], wrb[...], preferred_element_type=jnp.float32),
            0.0)
        hb[...] = hv
        tb[...] = jnp.dot(hv, wnb[...], preferred_element_type=jnp.float32)

    return pl.pallas_call(
        body,
        grid=(npad // blk,),
        in_specs=[
            pl.BlockSpec((NC, blk, fh), lambda i: (0, i, 0)),
            pl.BlockSpec((NS, blk), lambda i: (0, i)),
            pl.BlockSpec((blk, d), lambda i: (i, 0)),
            pl.BlockSpec((d, h), lambda i: (0, 0)),
            pl.BlockSpec((h, h), lambda i: (0, 0)),
            pl.BlockSpec((1, h), lambda i: (0, 0)),
        ],
        out_specs=[
            pl.BlockSpec((blk, h), lambda i: (i, 0)),
            pl.BlockSpec((blk, h), lambda i: (i, 0)),
        ],
        out_shape=[
            jax.ShapeDtypeStruct((npad, h), jnp.float32),
            jax.ShapeDtypeStruct((npad, h), jnp.float32),
        ],
    )(pa, cntf, xp, wr, wnext, b)


def _tc_final(pa, cntf, hp, wr, b, blk):
    npad, d = hp.shape
    h = wr.shape[1]
    fh = h // NC

    def body(pb, cb, xb, wrb, bb, ob):
        cw = cb[...]
        inv = 1.0 / jnp.maximum(jnp.sum(cw, axis=0), 1.0)
        p = pb[...]
        agg = jnp.concatenate([p[i] for i in range(NC)], axis=-1)
        ob[...] = (
            agg * inv[:, None] + bb[...]
            + jnp.dot(xb[...], wrb[...], preferred_element_type=jnp.float32))

    return pl.pallas_call(
        body,
        grid=(npad // blk,),
        in_specs=[
            pl.BlockSpec((NC, blk, fh), lambda i: (0, i, 0)),
            pl.BlockSpec((NS, blk), lambda i: (0, i)),
            pl.BlockSpec((blk, d), lambda i: (i, 0)),
            pl.BlockSpec((d, h), lambda i: (0, 0)),
            pl.BlockSpec((1, h), lambda i: (0, 0)),
        ],
        out_specs=pl.BlockSpec((blk, h), lambda i: (i, 0)),
        out_shape=jax.ShapeDtypeStruct((npad, h), jnp.float32),
    )(pa, cntf, hp, wr, b)


def _split_feats(t, npad):
    h = t.shape[1]
    fh = h // NC
    return jnp.stack([t[:, i * fh:(i + 1) * fh] for i in range(NC)])


def kernel(x, edge_index, W1l, b1l, W1r, W2l, b2l, W2r):
    n, d = x.shape
    h = W1l.shape[1]
    e = edge_index.shape[1]

    npad = _round_up(n + 1, 2048)
    cpt = _round_up(math.ceil(e / (NS * CH)), 8)
    epad = NS * cpt * CH
    blk = 256

    src = edge_index[0]
    dst = edge_index[1]
    if epad != e:
        pad = epad - e
        src = jnp.concatenate([src, jnp.zeros((pad,), jnp.int32)])
        dst = jnp.concatenate([dst, jnp.full((pad,), n, jnp.int32)])
    src2 = src.reshape(NS * cpt // SK, SK * CH)
    dst2 = dst.reshape(NS * cpt // SK, SK * CH)

    xp = jnp.zeros((npad, d), jnp.float32).at[:n].set(x)

    seg_c = _sc_segment_sum(npad, cpt, h, True)
    seg = _sc_segment_sum(npad, cpt, h, False)

    t1 = _tc_matmul(xp, W1l, blk)
    agg1, cnt_flat = seg_c(_split_feats(t1, npad), src2, dst2)
    cntf = cnt_flat.reshape(NS, npad)
    hp, t2 = _tc_combine(agg1, cntf, xp, W1r, W2l, b1l.reshape(1, h), blk)
    [agg2] = seg(_split_feats(t2, npad), src2, dst2)
    out = _tc_final(agg2, cntf, hp, W2r, b2l.reshape(1, h), blk)
    return out[:n]


# speedup vs baseline: 1.0410x; 1.0410x over previous
import functools
import math

import jax
import jax.numpy as jnp
from jax import lax
from jax.experimental import pallas as pl
from jax.experimental.pallas import tpu as pltpu
from jax.experimental.pallas import tpu_sc as plsc

NC = 2
NS = 16
L = 16
CH = 128


def _round_up(a, b):
    return (a + b - 1) // b * b


SK = 4


@functools.lru_cache(maxsize=None)
def _sc_segment_sum(npad, cpt, feat, with_count):
    fh = feat // NC
    rows_per_tile = npad // NS
    nz_full = rows_per_tile // CH
    z_rem = rows_per_tile - nz_full * CH
    mesh = plsc.VectorSubcoreMesh(core_axis_name="c", subcore_axis_name="s")

    out_type = [jax.ShapeDtypeStruct((NC, npad, fh), jnp.float32)]
    if with_count:
        out_type.append(jax.ShapeDtypeStruct((NS * npad,), jnp.float32))

    @functools.partial(
        pl.kernel,
        mesh=mesh,
        out_type=out_type,
        compiler_params=pltpu.CompilerParams(
            needs_layout_passes=False, use_tc_tiling_on_sc=False),
        scratch_types=[
            pltpu.VMEM((SK * CH,), jnp.int32),
            pltpu.VMEM((SK * CH,), jnp.int32),
            pltpu.VMEM((SK * CH,), jnp.int32),
            pltpu.VMEM((SK * CH,), jnp.int32),
            pltpu.VMEM((SK * CH, fh), jnp.float32),
            pltpu.VMEM((SK * CH, fh), jnp.float32),
            pltpu.VMEM((npad,), jnp.float32),
            pltpu.VMEM_SHARED((npad, fh), jnp.float32),
            pltpu.SemaphoreType.DMA,
            pltpu.SemaphoreType.DMA,
            pltpu.SemaphoreType.DMA,
            pltpu.SemaphoreType.DMA,
            pltpu.SemaphoreType.DMA,
            pltpu.SemaphoreType.DMA,
        ],
    )
    def seg(t_hbm, src_hbm, dst_hbm, *out_and_scratch):
        if with_count:
            out_hbm, cnt_hbm = out_and_scratch[:2]
            rest = out_and_scratch[2:]
        else:
            out_hbm = out_and_scratch[0]
            rest = out_and_scratch[1:]
        (s0, s1, d0, d1, rows0, rows1, cnt, acc,
         g0, g1, i0, i1, x0, x1) = rest
        sidxb = (s0, s1)
        didxb = (d0, d1)
        rows = (rows0, rows1)
        gsem = (g0, g1)
        isem = (i0, i1)
        ssem = (x0, x1)
        zbuf = rows0.at[pl.ds(0, CH)]
        c = lax.axis_index("c")
        s = lax.axis_index("s")
        row0 = s * rows_per_tile

        zv = jnp.zeros((L,), dtype=jnp.float32)

        def zrow(i, carry):
            for g in range(fh // L):
                zbuf[i, pl.ds(g * L, L)] = zv
            return carry

        lax.fori_loop(0, CH, zrow, 0)
        for z in range(nz_full):
            pltpu.sync_copy(zbuf, acc.at[pl.ds(row0 + z * CH, CH)])
        if z_rem:
            pltpu.sync_copy(zbuf.at[pl.ds(0, z_rem)],
                            acc.at[pl.ds(row0 + nz_full * CH, z_rem)])
        if with_count:
            def zi(i, carry):
                cnt[pl.ds(i * L, L)] = zv
                return carry

            lax.fori_loop(0, npad // L, zi, 0)
        plsc.subcore_barrier()

        ones = jnp.full((L,), 1.0, dtype=jnp.float32)
        nsuper = cpt // SK
        jrow0 = s * nsuper

        pltpu.sync_copy(src_hbm.at[jrow0], sidxb[0])
        pltpu.sync_copy(dst_hbm.at[jrow0], didxb[0])
        pltpu.async_copy(src_hbm.at[jrow0 + 1], sidxb[1], isem[1])
        pltpu.async_copy(dst_hbm.at[jrow0 + 1], didxb[1], isem[1])
        pltpu.async_copy(t_hbm.at[c].at[sidxb[0]], rows[0], gsem[0])

        def pair(p, carry):
            for b in range(2):
                j = 2 * p + b
                pltpu.make_async_copy(
                    t_hbm.at[c].at[sidxb[b]], rows[b], gsem[b]).wait()
                scat = pltpu.async_copy(
                    rows[b], acc.at[didxb[b]], ssem[b], add=True)
                nxt = j + 1

                @pl.when(nxt < nsuper)
                def _():
                    pltpu.make_async_copy(
                        src_hbm.at[jrow0 + nxt], sidxb[1 - b],
                        isem[1 - b]).wait()
                    pltpu.make_async_copy(
                        dst_hbm.at[jrow0 + nxt], didxb[1 - b],
                        isem[1 - b]).wait()
                    pltpu.async_copy(t_hbm.at[c].at[sidxb[1 - b]],
                                     rows[1 - b], gsem[1 - b])

                if with_count:
                    @pl.when(c == 0)
                    def _():
                        for i in range(SK * CH // L):
                            d = didxb[b][pl.ds(i * L, L)]
                            plsc.addupdate_scatter(cnt, (d,), ones)
                scat.wait()

                @pl.when(j + 2 < nsuper)
                def _():
                    pltpu.async_copy(src_hbm.at[jrow0 + j + 2],
                                     sidxb[b], isem[b])
                    pltpu.async_copy(dst_hbm.at[jrow0 + j + 2],
                                     didxb[b], isem[b])
            return carry

        lax.fori_loop(0, nsuper // 2, pair, 0)

        if with_count:
            @pl.when(c == 0)
            def _():
                pltpu.sync_copy(cnt, cnt_hbm.at[pl.ds(s * npad, npad)])
        plsc.subcore_barrier()

        pltpu.sync_copy(acc.at[pl.ds(row0, rows_per_tile)],
                        out_hbm.at[c].at[pl.ds(row0, rows_per_tile)])

    return seg


def _tc_matmul(xp, w, blk):
    npad, d = xp.shape
    h = w.shape[1]

    def body(xb, wb, ob):
        ob[...] = jnp.dot(xb[...], wb[...], preferred_element_type=jnp.float32)

    return pl.pallas_call(
        body,
        grid=(npad // blk,),
        in_specs=[
            pl.BlockSpec((blk, d), lambda i: (i, 0)),
            pl.BlockSpec((d, h), lambda i: (0, 0)),
        ],
        out_specs=pl.BlockSpec((blk, h), lambda i: (i, 0)),
        out_shape=jax.ShapeDtypeStruct((npad, h), jnp.float32),
    )(xp, w)


def _tc_combine(pa, cntf, xp, wr, wnext, b, blk):
    npad, d = xp.shape
    h = wr.shape[1]
    fh = h // NC

    def body(pb, cb, xb, wrb, wnb, bb, hb, tb):
        cw = cb[...]
        inv = 1.0 / jnp.maximum(jnp.sum(cw, axis=0), 1.0)
        p = pb[...]
        agg = jnp.concatenate([p[i] for i in range(NC)], axis=-1)
        hv = jnp.maximum(
            agg * inv[:, None] + bb[...]
            + jnp.dot(xb[...], wrb[...], preferred_element_type=jnp.float32),
            0.0)
        hb[...] = hv
        tb[...] = jnp.dot(hv, wnb[...], preferred_element_type=jnp.float32)

    return pl.pallas_call(
        body,
        grid=(npad // blk,),
        in_specs=[
            pl.BlockSpec((NC, blk, fh), lambda i: (0, i, 0)),
            pl.BlockSpec((NS, blk), lambda i: (0, i)),
            pl.BlockSpec((blk, d), lambda i: (i, 0)),
            pl.BlockSpec((d, h), lambda i: (0, 0)),
            pl.BlockSpec((h, h), lambda i: (0, 0)),
            pl.BlockSpec((1, h), lambda i: (0, 0)),
        ],
        out_specs=[
            pl.BlockSpec((blk, h), lambda i: (i, 0)),
            pl.BlockSpec((blk, h), lambda i: (i, 0)),
        ],
        out_shape=[
            jax.ShapeDtypeStruct((npad, h), jnp.float32),
            jax.ShapeDtypeStruct((npad, h), jnp.float32),
        ],
    )(pa, cntf, xp, wr, wnext, b)


def _tc_final(pa, cntf, hp, wr, b, blk):
    npad, d = hp.shape
    h = wr.shape[1]
    fh = h // NC

    def body(pb, cb, xb, wrb, bb, ob):
        cw = cb[...]
        inv = 1.0 / jnp.maximum(jnp.sum(cw, axis=0), 1.0)
        p = pb[...]
        agg = jnp.concatenate([p[i] for i in range(NC)], axis=-1)
        ob[...] = (
            agg * inv[:, None] + bb[...]
            + jnp.dot(xb[...], wrb[...], preferred_element_type=jnp.float32))

    return pl.pallas_call(
        body,
        grid=(npad // blk,),
        in_specs=[
            pl.BlockSpec((NC, blk, fh), lambda i: (0, i, 0)),
            pl.BlockSpec((NS, blk), lambda i: (0, i)),
            pl.BlockSpec((blk, d), lambda i: (i, 0)),
            pl.BlockSpec((d, h), lambda i: (0, 0)),
            pl.BlockSpec((1, h), lambda i: (0, 0)),
        ],
        out_specs=pl.BlockSpec((blk, h), lambda i: (i, 0)),
        out_shape=jax.ShapeDtypeStruct((npad, h), jnp.float32),
    )(pa, cntf, hp, wr, b)


def _split_feats(t, npad):
    h = t.shape[1]
    fh = h // NC
    return jnp.stack([t[:, i * fh:(i + 1) * fh] for i in range(NC)])


def kernel(x, edge_index, W1l, b1l, W1r, W2l, b2l, W2r):
    n, d = x.shape
    h = W1l.shape[1]
    e = edge_index.shape[1]

    npad = _round_up(n + 1, 2048)
    cpt = _round_up(math.ceil(e / (NS * CH)), 8)
    epad = NS * cpt * CH
    blk = 512

    src = edge_index[0]
    dst = edge_index[1]
    if epad != e:
        pad = epad - e
        src = jnp.concatenate([src, jnp.zeros((pad,), jnp.int32)])
        dst = jnp.concatenate([dst, jnp.full((pad,), n, jnp.int32)])
    src2 = src.reshape(NS * cpt // SK, SK * CH)
    dst2 = dst.reshape(NS * cpt // SK, SK * CH)

    xp = jnp.zeros((npad, d), jnp.float32).at[:n].set(x)

    seg_c = _sc_segment_sum(npad, cpt, h, True)
    seg = _sc_segment_sum(npad, cpt, h, False)

    t1 = _tc_matmul(xp, W1l, blk)
    agg1, cnt_flat = seg_c(_split_feats(t1, npad), src2, dst2)
    cntf = cnt_flat.reshape(NS, npad)
    hp, t2 = _tc_combine(agg1, cntf, xp, W1r, W2l, b1l.reshape(1, h), blk)
    [agg2] = seg(_split_feats(t2, npad), src2, dst2)
    out = _tc_final(agg2, cntf, hp, W2r, b2l.reshape(1, h), blk)
    return out[:n]


# speedup vs baseline: 1.0646x; 1.0227x over previous
import functools
import math

import jax
import jax.numpy as jnp
from jax import lax
from jax.experimental import pallas as pl
from jax.experimental.pallas import tpu as pltpu
from jax.experimental.pallas import tpu_sc as plsc

NC = 2
NS = 16
L = 16
CH = 128


def _round_up(a, b):
    return (a + b - 1) // b * b


SK = 4


@functools.lru_cache(maxsize=None)
def _sc_segment_sum(npad, cpt, feat, with_count):
    fh = feat // NC
    rows_per_tile = npad // NS
    nz_full = rows_per_tile // CH
    z_rem = rows_per_tile - nz_full * CH
    mesh = plsc.VectorSubcoreMesh(core_axis_name="c", subcore_axis_name="s")

    out_type = [jax.ShapeDtypeStruct((NC, npad, fh), jnp.float32)]
    if with_count:
        out_type.append(jax.ShapeDtypeStruct((NS * npad,), jnp.float32))

    @functools.partial(
        pl.kernel,
        mesh=mesh,
        out_type=out_type,
        compiler_params=pltpu.CompilerParams(
            needs_layout_passes=False, use_tc_tiling_on_sc=False),
        scratch_types=[
            pltpu.VMEM((SK * CH,), jnp.int32),
            pltpu.VMEM((SK * CH,), jnp.int32),
            pltpu.VMEM((SK * CH,), jnp.int32),
            pltpu.VMEM((SK * CH,), jnp.int32),
            pltpu.VMEM((SK * CH, fh), jnp.float32),
            pltpu.VMEM((SK * CH, fh), jnp.float32),
            pltpu.VMEM((npad,), jnp.float32),
            pltpu.VMEM_SHARED((npad, fh), jnp.float32),
            pltpu.SemaphoreType.DMA,
            pltpu.SemaphoreType.DMA,
            pltpu.SemaphoreType.DMA,
            pltpu.SemaphoreType.DMA,
            pltpu.SemaphoreType.DMA,
            pltpu.SemaphoreType.DMA,
        ],
    )
    def seg(t_hbm, src_hbm, dst_hbm, *out_and_scratch):
        if with_count:
            out_hbm, cnt_hbm = out_and_scratch[:2]
            rest = out_and_scratch[2:]
        else:
            out_hbm = out_and_scratch[0]
            rest = out_and_scratch[1:]
        (s0, s1, d0, d1, rows0, rows1, cnt, acc,
         g0, g1, i0, i1, x0, x1) = rest
        sidxb = (s0, s1)
        didxb = (d0, d1)
        rows = (rows0, rows1)
        gsem = (g0, g1)
        isem = (i0, i1)
        ssem = (x0, x1)
        zbuf = rows0.at[pl.ds(0, CH)]
        c = lax.axis_index("c")
        s = lax.axis_index("s")
        row0 = s * rows_per_tile

        zv = jnp.zeros((L,), dtype=jnp.float32)

        def zrow(i, carry):
            for g in range(fh // L):
                zbuf[i, pl.ds(g * L, L)] = zv
            return carry

        lax.fori_loop(0, CH, zrow, 0)
        for z in range(nz_full):
            pltpu.sync_copy(zbuf, acc.at[pl.ds(row0 + z * CH, CH)])
        if z_rem:
            pltpu.sync_copy(zbuf.at[pl.ds(0, z_rem)],
                            acc.at[pl.ds(row0 + nz_full * CH, z_rem)])
        if with_count:
            def zi(i, carry):
                cnt[pl.ds(i * L, L)] = zv
                return carry

            lax.fori_loop(0, npad // L, zi, 0)
        plsc.subcore_barrier()

        ones = jnp.full((L,), 1.0, dtype=jnp.float32)
        nsuper = cpt // SK
        jrow0 = s * nsuper

        pltpu.sync_copy(src_hbm.at[jrow0], sidxb[0])
        pltpu.sync_copy(dst_hbm.at[jrow0], didxb[0])
        pltpu.async_copy(src_hbm.at[jrow0 + 1], sidxb[1], isem[1])
        pltpu.async_copy(dst_hbm.at[jrow0 + 1], didxb[1], isem[1])
        pltpu.async_copy(t_hbm.at[c].at[sidxb[0]], rows[0], gsem[0])

        def pair(p, carry):
            for b in range(2):
                j = 2 * p + b
                pltpu.make_async_copy(
                    t_hbm.at[c].at[sidxb[b]], rows[b], gsem[b]).wait()
                scat = pltpu.async_copy(
                    rows[b], acc.at[didxb[b]], ssem[b], add=True)
                nxt = j + 1

                @pl.when(nxt < nsuper)
                def _():
                    pltpu.make_async_copy(
                        src_hbm.at[jrow0 + nxt], sidxb[1 - b],
                        isem[1 - b]).wait()
                    pltpu.make_async_copy(
                        dst_hbm.at[jrow0 + nxt], didxb[1 - b],
                        isem[1 - b]).wait()
                    pltpu.async_copy(t_hbm.at[c].at[sidxb[1 - b]],
                                     rows[1 - b], gsem[1 - b])

                if with_count:
                    @pl.when(c == 0)
                    def _():
                        for i in range(SK * CH // L):
                            d = didxb[b][pl.ds(i * L, L)]
                            plsc.addupdate_scatter(cnt, (d,), ones)
                scat.wait()

                @pl.when(j + 2 < nsuper)
                def _():
                    pltpu.async_copy(src_hbm.at[jrow0 + j + 2],
                                     sidxb[b], isem[b])
                    pltpu.async_copy(dst_hbm.at[jrow0 + j + 2],
                                     didxb[b], isem[b])
            return carry

        lax.fori_loop(0, nsuper // 2, pair, 0)

        if with_count:
            @pl.when(c == 0)
            def _():
                pltpu.sync_copy(cnt, cnt_hbm.at[pl.ds(s * npad, npad)])
        plsc.subcore_barrier()

        pltpu.sync_copy(acc.at[pl.ds(row0, rows_per_tile)],
                        out_hbm.at[c].at[pl.ds(row0, rows_per_tile)])

    return seg


def _tc_matmul(xp, w, blk):
    npad, d = xp.shape
    h = w.shape[1]

    def body(xb, wb, ob):
        ob[...] = jnp.dot(xb[...], wb[...], preferred_element_type=jnp.float32)

    return pl.pallas_call(
        body,
        grid=(npad // blk,),
        in_specs=[
            pl.BlockSpec((blk, d), lambda i: (i, 0)),
            pl.BlockSpec((d, h), lambda i: (0, 0)),
        ],
        out_specs=pl.BlockSpec((blk, h), lambda i: (i, 0)),
        out_shape=jax.ShapeDtypeStruct((npad, h), jnp.float32),
    )(xp, w)


def _tc_combine(pa, cntf, xp, wr, wnext, b, blk):
    npad, d = xp.shape
    h = wr.shape[1]
    fh = h // NC

    def body(pb, cb, xb, wrb, wnb, bb, hb, tb):
        cw = cb[...]
        inv = 1.0 / jnp.maximum(jnp.sum(cw, axis=0), 1.0)
        p = pb[...]
        agg = jnp.concatenate([p[i] for i in range(NC)], axis=-1)
        hv = jnp.maximum(
            agg * inv[:, None] + bb[...]
            + jnp.dot(xb[...], wrb[...], preferred_element_type=jnp.float32),
            0.0)
        hb[...] = hv
        tb[...] = jnp.dot(hv, wnb[...], preferred_element_type=jnp.float32)

    return pl.pallas_call(
        body,
        grid=(npad // blk,),
        in_specs=[
            pl.BlockSpec((NC, blk, fh), lambda i: (0, i, 0)),
            pl.BlockSpec((NS, blk), lambda i: (0, i)),
            pl.BlockSpec((blk, d), lambda i: (i, 0)),
            pl.BlockSpec((d, h), lambda i: (0, 0)),
            pl.BlockSpec((h, h), lambda i: (0, 0)),
            pl.BlockSpec((1, h), lambda i: (0, 0)),
        ],
        out_specs=[
            pl.BlockSpec((blk, h), lambda i: (i, 0)),
            pl.BlockSpec((blk, h), lambda i: (i, 0)),
        ],
        out_shape=[
            jax.ShapeDtypeStruct((npad, h), jnp.float32),
            jax.ShapeDtypeStruct((npad, h), jnp.float32),
        ],
    )(pa, cntf, xp, wr, wnext, b)


def _tc_final(pa, cntf, hp, wr, b, blk):
    npad, d = hp.shape
    h = wr.shape[1]
    fh = h // NC

    def body(pb, cb, xb, wrb, bb, ob):
        cw = cb[...]
        inv = 1.0 / jnp.maximum(jnp.sum(cw, axis=0), 1.0)
        p = pb[...]
        agg = jnp.concatenate([p[i] for i in range(NC)], axis=-1)
        ob[...] = (
            agg * inv[:, None] + bb[...]
            + jnp.dot(xb[...], wrb[...], preferred_element_type=jnp.float32))

    return pl.pallas_call(
        body,
        grid=(npad // blk,),
        in_specs=[
            pl.BlockSpec((NC, blk, fh), lambda i: (0, i, 0)),
            pl.BlockSpec((NS, blk), lambda i: (0, i)),
            pl.BlockSpec((blk, d), lambda i: (i, 0)),
            pl.BlockSpec((d, h), lambda i: (0, 0)),
            pl.BlockSpec((1, h), lambda i: (0, 0)),
        ],
        out_specs=pl.BlockSpec((blk, h), lambda i: (i, 0)),
        out_shape=jax.ShapeDtypeStruct((npad, h), jnp.float32),
    )(pa, cntf, hp, wr, b)


def _split_feats(t, npad):
    h = t.shape[1]
    fh = h // NC
    return jnp.stack([t[:, i * fh:(i + 1) * fh] for i in range(NC)])


def kernel(x, edge_index, W1l, b1l, W1r, W2l, b2l, W2r):
    n, d = x.shape
    h = W1l.shape[1]
    e = edge_index.shape[1]

    npad = _round_up(n + 1, 2048)
    cpt = _round_up(math.ceil(e / (NS * CH)), 8)
    epad = NS * cpt * CH
    blk = 1024

    src = edge_index[0]
    dst = edge_index[1]
    if epad != e:
        pad = epad - e
        src = jnp.concatenate([src, jnp.zeros((pad,), jnp.int32)])
        dst = jnp.concatenate([dst, jnp.full((pad,), n, jnp.int32)])
    src2 = src.reshape(NS * cpt // SK, SK * CH)
    dst2 = dst.reshape(NS * cpt // SK, SK * CH)

    xp = jnp.zeros((npad, d), jnp.float32).at[:n].set(x)

    seg_c = _sc_segment_sum(npad, cpt, h, True)
    seg = _sc_segment_sum(npad, cpt, h, False)

    t1 = _tc_matmul(xp, W1l, blk)
    agg1, cnt_flat = seg_c(_split_feats(t1, npad), src2, dst2)
    cntf = cnt_flat.reshape(NS, npad)
    hp, t2 = _tc_combine(agg1, cntf, xp, W1r, W2l, b1l.reshape(1, h), blk)
    [agg2] = seg(_split_feats(t2, npad), src2, dst2)
    out = _tc_final(agg2, cntf, hp, W2r, b2l.reshape(1, h), blk)
    return out[:n]


# speedup vs baseline: 1.0720x; 1.0070x over previous
import functools
import math

import jax
import jax.numpy as jnp
from jax import lax
from jax.experimental import pallas as pl
from jax.experimental.pallas import tpu as pltpu
from jax.experimental.pallas import tpu_sc as plsc

NC = 2
NS = 16
L = 16
CH = 128


def _round_up(a, b):
    return (a + b - 1) // b * b


SK = 4


@functools.lru_cache(maxsize=None)
def _sc_segment_sum(npad, cpt, feat, with_count):
    fh = feat // NC
    rows_per_tile = npad // NS
    nz_full = rows_per_tile // CH
    z_rem = rows_per_tile - nz_full * CH
    mesh = plsc.VectorSubcoreMesh(core_axis_name="c", subcore_axis_name="s")

    out_type = [jax.ShapeDtypeStruct((NC, npad, fh), jnp.float32)]
    if with_count:
        out_type.append(jax.ShapeDtypeStruct((NS * npad,), jnp.float32))

    @functools.partial(
        pl.kernel,
        mesh=mesh,
        out_type=out_type,
        compiler_params=pltpu.CompilerParams(
            needs_layout_passes=False, use_tc_tiling_on_sc=False),
        scratch_types=[
            pltpu.VMEM((SK * CH,), jnp.int32),
            pltpu.VMEM((SK * CH,), jnp.int32),
            pltpu.VMEM((SK * CH,), jnp.int32),
            pltpu.VMEM((SK * CH,), jnp.int32),
            pltpu.VMEM((SK * CH, fh), jnp.float32),
            pltpu.VMEM((SK * CH, fh), jnp.float32),
            pltpu.VMEM((npad,), jnp.float32),
            pltpu.VMEM_SHARED((npad, fh), jnp.float32),
            pltpu.SemaphoreType.DMA,
            pltpu.SemaphoreType.DMA,
            pltpu.SemaphoreType.DMA,
            pltpu.SemaphoreType.DMA,
            pltpu.SemaphoreType.DMA,
            pltpu.SemaphoreType.DMA,
        ],
    )
    def seg(t_hbm, src_hbm, dst_hbm, *out_and_scratch):
        if with_count:
            out_hbm, cnt_hbm = out_and_scratch[:2]
            rest = out_and_scratch[2:]
        else:
            out_hbm = out_and_scratch[0]
            rest = out_and_scratch[1:]
        (s0, s1, d0, d1, rows0, rows1, cnt, acc,
         g0, g1, i0, i1, x0, x1) = rest
        sidxb = (s0, s1)
        didxb = (d0, d1)
        rows = (rows0, rows1)
        gsem = (g0, g1)
        isem = (i0, i1)
        ssem = (x0, x1)
        zbuf = rows0.at[pl.ds(0, CH)]
        c = lax.axis_index("c")
        s = lax.axis_index("s")
        row0 = s * rows_per_tile

        zv = jnp.zeros((L,), dtype=jnp.float32)

        def zrow(i, carry):
            for g in range(fh // L):
                zbuf[i, pl.ds(g * L, L)] = zv
            return carry

        lax.fori_loop(0, CH, zrow, 0)
        for z in range(nz_full):
            pltpu.sync_copy(zbuf, acc.at[pl.ds(row0 + z * CH, CH)])
        if z_rem:
            pltpu.sync_copy(zbuf.at[pl.ds(0, z_rem)],
                            acc.at[pl.ds(row0 + nz_full * CH, z_rem)])
        if with_count:
            def zi(i, carry):
                cnt[pl.ds(i * L, L)] = zv
                return carry

            lax.fori_loop(0, npad // L, zi, 0)
        plsc.subcore_barrier()

        ones = jnp.full((L,), 1.0, dtype=jnp.float32)
        nsuper = cpt // SK
        jrow0 = s * nsuper

        pltpu.sync_copy(src_hbm.at[jrow0], sidxb[0])
        pltpu.sync_copy(dst_hbm.at[jrow0], didxb[0])
        pltpu.async_copy(src_hbm.at[jrow0 + 1], sidxb[1], isem[1])
        pltpu.async_copy(dst_hbm.at[jrow0 + 1], didxb[1], isem[1])
        pltpu.async_copy(t_hbm.at[c].at[sidxb[0]], rows[0], gsem[0])

        def pair(p, carry):
            for b in range(2):
                j = 2 * p + b
                pltpu.make_async_copy(
                    t_hbm.at[c].at[sidxb[b]], rows[b], gsem[b]).wait()
                scat = pltpu.async_copy(
                    rows[b], acc.at[didxb[b]], ssem[b], add=True)
                nxt = j + 1

                @pl.when(nxt < nsuper)
                def _():
                    pltpu.make_async_copy(
                        src_hbm.at[jrow0 + nxt], sidxb[1 - b],
                        isem[1 - b]).wait()
                    pltpu.make_async_copy(
                        dst_hbm.at[jrow0 + nxt], didxb[1 - b],
                        isem[1 - b]).wait()
                    pltpu.async_copy(t_hbm.at[c].at[sidxb[1 - b]],
                                     rows[1 - b], gsem[1 - b])

                if with_count:
                    @pl.when(c == 0)
                    def _():
                        for i in range(SK * CH // L):
                            d = didxb[b][pl.ds(i * L, L)]
                            plsc.addupdate_scatter(cnt, (d,), ones)
                scat.wait()

                @pl.when(j + 2 < nsuper)
                def _():
                    pltpu.async_copy(src_hbm.at[jrow0 + j + 2],
                                     sidxb[b], isem[b])
                    pltpu.async_copy(dst_hbm.at[jrow0 + j + 2],
                                     didxb[b], isem[b])
            return carry

        lax.fori_loop(0, nsuper // 2, pair, 0)

        if with_count:
            @pl.when(c == 0)
            def _():
                pltpu.sync_copy(cnt, cnt_hbm.at[pl.ds(s * npad, npad)])
        plsc.subcore_barrier()

        pltpu.sync_copy(acc.at[pl.ds(row0, rows_per_tile)],
                        out_hbm.at[c].at[pl.ds(row0, rows_per_tile)])

    return seg


def _tc_matmul(xp, w, blk):
    npad, d = xp.shape
    h = w.shape[1]

    def body(xb, wb, ob):
        ob[...] = jnp.dot(xb[...], wb[...], preferred_element_type=jnp.float32)

    return pl.pallas_call(
        body,
        grid=(npad // blk,),
        in_specs=[
            pl.BlockSpec((blk, d), lambda i: (i, 0)),
            pl.BlockSpec((d, h), lambda i: (0, 0)),
        ],
        out_specs=pl.BlockSpec((blk, h), lambda i: (i, 0)),
        out_shape=jax.ShapeDtypeStruct((npad, h), jnp.float32),
    )(xp, w)


def _tc_combine(pa, cntf, xp, wr, wnext, b, blk):
    npad, d = xp.shape
    h = wr.shape[1]
    fh = h // NC

    def body(pb, cb, xb, wrb, wnb, bb, hb, tb):
        cw = cb[...]
        inv = 1.0 / jnp.maximum(jnp.sum(cw, axis=0), 1.0)
        p = pb[...]
        agg = jnp.concatenate([p[i] for i in range(NC)], axis=-1)
        hv = jnp.maximum(
            agg * inv[:, None] + bb[...]
            + jnp.dot(xb[...], wrb[...], preferred_element_type=jnp.float32),
            0.0)
        hb[...] = hv
        tb[...] = jnp.dot(hv, wnb[...], preferred_element_type=jnp.float32)

    return pl.pallas_call(
        body,
        grid=(npad // blk,),
        in_specs=[
            pl.BlockSpec((NC, blk, fh), lambda i: (0, i, 0)),
            pl.BlockSpec((NS, blk), lambda i: (0, i)),
            pl.BlockSpec((blk, d), lambda i: (i, 0)),
            pl.BlockSpec((d, h), lambda i: (0, 0)),
            pl.BlockSpec((h, h), lambda i: (0, 0)),
            pl.BlockSpec((1, h), lambda i: (0, 0)),
        ],
        out_specs=[
            pl.BlockSpec((blk, h), lambda i: (i, 0)),
            pl.BlockSpec((blk, h), lambda i: (i, 0)),
        ],
        out_shape=[
            jax.ShapeDtypeStruct((npad, h), jnp.float32),
            jax.ShapeDtypeStruct((npad, h), jnp.float32),
        ],
    )(pa, cntf, xp, wr, wnext, b)


def _tc_final(pa, cntf, hp, wr, b, blk):
    npad, d = hp.shape
    h = wr.shape[1]
    fh = h // NC

    def body(pb, cb, xb, wrb, bb, ob):
        cw = cb[...]
        inv = 1.0 / jnp.maximum(jnp.sum(cw, axis=0), 1.0)
        p = pb[...]
        agg = jnp.concatenate([p[i] for i in range(NC)], axis=-1)
        ob[...] = (
            agg * inv[:, None] + bb[...]
            + jnp.dot(xb[...], wrb[...], preferred_element_type=jnp.float32))

    return pl.pallas_call(
        body,
        grid=(npad // blk,),
        in_specs=[
            pl.BlockSpec((NC, blk, fh), lambda i: (0, i, 0)),
            pl.BlockSpec((NS, blk), lambda i: (0, i)),
            pl.BlockSpec((blk, d), lambda i: (i, 0)),
            pl.BlockSpec((d, h), lambda i: (0, 0)),
            pl.BlockSpec((1, h), lambda i: (0, 0)),
        ],
        out_specs=pl.BlockSpec((blk, h), lambda i: (i, 0)),
        out_shape=jax.ShapeDtypeStruct((npad, h), jnp.float32),
    )(pa, cntf, hp, wr, b)


def _split_feats(t, npad):
    h = t.shape[1]
    fh = h // NC
    return jnp.stack([t[:, i * fh:(i + 1) * fh] for i in range(NC)])


def kernel(x, edge_index, W1l, b1l, W1r, W2l, b2l, W2r):
    n, d = x.shape
    h = W1l.shape[1]
    e = edge_index.shape[1]

    npad = _round_up(n + 1, 2048)
    cpt = _round_up(math.ceil(e / (NS * CH)), 8)
    epad = NS * cpt * CH
    blk = 2048

    src = edge_index[0]
    dst = edge_index[1]
    if epad != e:
        pad = epad - e
        src = jnp.concatenate([src, jnp.zeros((pad,), jnp.int32)])
        dst = jnp.concatenate([dst, jnp.full((pad,), n, jnp.int32)])
    src2 = src.reshape(NS * cpt // SK, SK * CH)
    dst2 = dst.reshape(NS * cpt // SK, SK * CH)

    xp = jnp.zeros((npad, d), jnp.float32).at[:n].set(x)

    seg_c = _sc_segment_sum(npad, cpt, h, True)
    seg = _sc_segment_sum(npad, cpt, h, False)

    t1 = _tc_matmul(xp, W1l, blk)
    agg1, cnt_flat = seg_c(_split_feats(t1, npad), src2, dst2)
    cntf = cnt_flat.reshape(NS, npad)
    hp, t2 = _tc_combine(agg1, cntf, xp, W1r, W2l, b1l.reshape(1, h), blk)
    [agg2] = seg(_split_feats(t2, npad), src2, dst2)
    out = _tc_final(agg2, cntf, hp, W2r, b2l.reshape(1, h), blk)
    return out[:n]


# speedup vs baseline: 1.0828x; 1.0101x over previous
import functools
import math

import jax
import jax.numpy as jnp
from jax import lax
from jax.experimental import pallas as pl
from jax.experimental.pallas import tpu as pltpu
from jax.experimental.pallas import tpu_sc as plsc

NC = 2
NS = 16
L = 16
CH = 128


def _round_up(a, b):
    return (a + b - 1) // b * b


SK = 4


@functools.lru_cache(maxsize=None)
def _sc_segment_sum(npad, cpt, feat, with_count):
    fh = feat // NC
    rows_per_tile = npad // NS
    nz_full = rows_per_tile // CH
    z_rem = rows_per_tile - nz_full * CH
    mesh = plsc.VectorSubcoreMesh(core_axis_name="c", subcore_axis_name="s")

    out_type = [jax.ShapeDtypeStruct((NC, npad, fh), jnp.float32)]
    if with_count:
        out_type.append(jax.ShapeDtypeStruct((NS * npad,), jnp.float32))

    @functools.partial(
        pl.kernel,
        mesh=mesh,
        out_type=out_type,
        compiler_params=pltpu.CompilerParams(
            needs_layout_passes=False, use_tc_tiling_on_sc=False),
        scratch_types=[
            pltpu.VMEM((SK * CH,), jnp.int32),
            pltpu.VMEM((SK * CH,), jnp.int32),
            pltpu.VMEM((SK * CH,), jnp.int32),
            pltpu.VMEM((SK * CH,), jnp.int32),
            pltpu.VMEM((SK * CH, fh), jnp.float32),
            pltpu.VMEM((SK * CH, fh), jnp.float32),
            pltpu.VMEM((npad,), jnp.float32),
            pltpu.VMEM_SHARED((npad, fh), jnp.float32),
            pltpu.SemaphoreType.DMA,
            pltpu.SemaphoreType.DMA,
            pltpu.SemaphoreType.DMA,
            pltpu.SemaphoreType.DMA,
            pltpu.SemaphoreType.DMA,
            pltpu.SemaphoreType.DMA,
        ],
    )
    def seg(t_hbm, src_hbm, dst_hbm, *out_and_scratch):
        if with_count:
            out_hbm, cnt_hbm = out_and_scratch[:2]
            rest = out_and_scratch[2:]
        else:
            out_hbm = out_and_scratch[0]
            rest = out_and_scratch[1:]
        (s0, s1, d0, d1, rows0, rows1, cnt, acc,
         g0, g1, i0, i1, x0, x1) = rest
        sidxb = (s0, s1)
        didxb = (d0, d1)
        rows = (rows0, rows1)
        gsem = (g0, g1)
        isem = (i0, i1)
        ssem = (x0, x1)
        zbuf = rows0.at[pl.ds(0, CH)]
        c = lax.axis_index("c")
        s = lax.axis_index("s")
        row0 = s * rows_per_tile

        zv = jnp.zeros((L,), dtype=jnp.float32)

        def zrow(i, carry):
            for g in range(fh // L):
                zbuf[i, pl.ds(g * L, L)] = zv
            return carry

        lax.fori_loop(0, CH, zrow, 0)
        for z in range(nz_full):
            pltpu.sync_copy(zbuf, acc.at[pl.ds(row0 + z * CH, CH)])
        if z_rem:
            pltpu.sync_copy(zbuf.at[pl.ds(0, z_rem)],
                            acc.at[pl.ds(row0 + nz_full * CH, z_rem)])
        if with_count:
            def zi(i, carry):
                cnt[pl.ds(i * L, L)] = zv
                return carry

            lax.fori_loop(0, npad // L, zi, 0)
        plsc.subcore_barrier()

        ones = jnp.full((L,), 1.0, dtype=jnp.float32)
        nsuper = cpt // SK
        jrow0 = s * nsuper

        pltpu.sync_copy(src_hbm.at[jrow0], sidxb[0])
        pltpu.sync_copy(dst_hbm.at[jrow0], didxb[0])
        pltpu.async_copy(src_hbm.at[jrow0 + 1], sidxb[1], isem[1])
        pltpu.async_copy(dst_hbm.at[jrow0 + 1], didxb[1], isem[1])
        pltpu.async_copy(t_hbm.at[c].at[sidxb[0]], rows[0], gsem[0])

        def pair(p, carry):
            for b in range(2):
                j = 2 * p + b
                pltpu.make_async_copy(
                    t_hbm.at[c].at[sidxb[b]], rows[b], gsem[b]).wait()
                scat = pltpu.async_copy(
                    rows[b], acc.at[didxb[b]], ssem[b], add=True)
                nxt = j + 1

                @pl.when(nxt < nsuper)
                def _():
                    pltpu.make_async_copy(
                        src_hbm.at[jrow0 + nxt], sidxb[1 - b],
                        isem[1 - b]).wait()
                    pltpu.make_async_copy(
                        dst_hbm.at[jrow0 + nxt], didxb[1 - b],
                        isem[1 - b]).wait()
                    pltpu.async_copy(t_hbm.at[c].at[sidxb[1 - b]],
                                     rows[1 - b], gsem[1 - b])

                if with_count:
                    @pl.when(c == 0)
                    def _():
                        for i in range(SK * CH // L):
                            d = didxb[b][pl.ds(i * L, L)]
                            plsc.addupdate_scatter(cnt, (d,), ones)
                scat.wait()

                @pl.when(j + 2 < nsuper)
                def _():
                    pltpu.async_copy(src_hbm.at[jrow0 + j + 2],
                                     sidxb[b], isem[b])
                    pltpu.async_copy(dst_hbm.at[jrow0 + j + 2],
                                     didxb[b], isem[b])
            return carry

        lax.fori_loop(0, nsuper // 2, pair, 0)

        if with_count:
            @pl.when(c == 0)
            def _():
                pltpu.sync_copy(cnt, cnt_hbm.at[pl.ds(s * npad, npad)])
        plsc.subcore_barrier()

        pltpu.sync_copy(acc.at[pl.ds(row0, rows_per_tile)],
                        out_hbm.at[c].at[pl.ds(row0, rows_per_tile)])

    return seg


def _tc_matmul(xp, w, blk):
    npad, d = xp.shape
    h = w.shape[1]

    def body(xb, wb, ob):
        ob[...] = jnp.dot(xb[...], wb[...], preferred_element_type=jnp.float32)

    return pl.pallas_call(
        body,
        grid=(npad // blk,),
        in_specs=[
            pl.BlockSpec((blk, d), lambda i: (i, 0)),
            pl.BlockSpec((d, h), lambda i: (0, 0)),
        ],
        out_specs=pl.BlockSpec((blk, h), lambda i: (i, 0)),
        out_shape=jax.ShapeDtypeStruct((npad, h), jnp.float32),
    )(xp, w)


def _tc_combine(pa, cntf, xp, wr, wnext, b, blk):
    npad, d = xp.shape
    h = wr.shape[1]
    fh = h // NC

    def body(pb, cb, xb, wrb, wnb, bb, hb, tb):
        cw = cb[...]
        inv = 1.0 / jnp.maximum(jnp.sum(cw, axis=0), 1.0)
        p = pb[...]
        agg = jnp.concatenate([p[i] for i in range(NC)], axis=-1)
        hv = jnp.maximum(
            agg * inv[:, None] + bb[...]
            + jnp.dot(xb[...], wrb[...], preferred_element_type=jnp.float32),
            0.0)
        hb[...] = hv
        tb[...] = jnp.dot(hv, wnb[...], preferred_element_type=jnp.float32)

    return pl.pallas_call(
        body,
        grid=(npad // blk,),
        in_specs=[
            pl.BlockSpec((NC, blk, fh), lambda i: (0, i, 0)),
            pl.BlockSpec((NS, blk), lambda i: (0, i)),
            pl.BlockSpec((blk, d), lambda i: (i, 0)),
            pl.BlockSpec((d, h), lambda i: (0, 0)),
            pl.BlockSpec((h, h), lambda i: (0, 0)),
            pl.BlockSpec((1, h), lambda i: (0, 0)),
        ],
        out_specs=[
            pl.BlockSpec((blk, h), lambda i: (i, 0)),
            pl.BlockSpec((blk, h), lambda i: (i, 0)),
        ],
        out_shape=[
            jax.ShapeDtypeStruct((npad, h), jnp.float32),
            jax.ShapeDtypeStruct((npad, h), jnp.float32),
        ],
    )(pa, cntf, xp, wr, wnext, b)


def _tc_final(pa, cntf, hp, wr, b, blk):
    npad, d = hp.shape
    h = wr.shape[1]
    fh = h // NC

    def body(pb, cb, xb, wrb, bb, ob):
        cw = cb[...]
        inv = 1.0 / jnp.maximum(jnp.sum(cw, axis=0), 1.0)
        p = pb[...]
        agg = jnp.concatenate([p[i] for i in range(NC)], axis=-1)
        ob[...] = (
            agg * inv[:, None] + bb[...]
            + jnp.dot(xb[...], wrb[...], preferred_element_type=jnp.float32))

    return pl.pallas_call(
        body,
        grid=(npad // blk,),
        in_specs=[
            pl.BlockSpec((NC, blk, fh), lambda i: (0, i, 0)),
            pl.BlockSpec((NS, blk), lambda i: (0, i)),
            pl.BlockSpec((blk, d), lambda i: (i, 0)),
            pl.BlockSpec((d, h), lambda i: (0, 0)),
            pl.BlockSpec((1, h), lambda i: (0, 0)),
        ],
        out_specs=pl.BlockSpec((blk, h), lambda i: (i, 0)),
        out_shape=jax.ShapeDtypeStruct((npad, h), jnp.float32),
    )(pa, cntf, hp, wr, b)


def _split_feats(t, npad):
    h = t.shape[1]
    fh = h // NC
    return jnp.stack([t[:, i * fh:(i + 1) * fh] for i in range(NC)])


def kernel(x, edge_index, W1l, b1l, W1r, W2l, b2l, W2r):
    n, d = x.shape
    h = W1l.shape[1]
    e = edge_index.shape[1]

    npad = _round_up(n + 1, 2048)
    cpt = _round_up(math.ceil(e / (NS * CH)), 8)
    epad = NS * cpt * CH
    blk = 5120

    src = edge_index[0]
    dst = edge_index[1]
    if epad != e:
        pad = epad - e
        src = jnp.concatenate([src, jnp.zeros((pad,), jnp.int32)])
        dst = jnp.concatenate([dst, jnp.full((pad,), n, jnp.int32)])
    src2 = src.reshape(NS * cpt // SK, SK * CH)
    dst2 = dst.reshape(NS * cpt // SK, SK * CH)

    xp = jnp.zeros((npad, d), jnp.float32).at[:n].set(x)

    seg_c = _sc_segment_sum(npad, cpt, h, True)
    seg = _sc_segment_sum(npad, cpt, h, False)

    t1 = _tc_matmul(xp, W1l, blk)
    agg1, cnt_flat = seg_c(_split_feats(t1, npad), src2, dst2)
    cntf = cnt_flat.reshape(NS, npad)
    hp, t2 = _tc_combine(agg1, cntf, xp, W1r, W2l, b1l.reshape(1, h), blk)
    [agg2] = seg(_split_feats(t2, npad), src2, dst2)
    out = _tc_final(agg2, cntf, hp, W2r, b2l.reshape(1, h), blk)
    return out[:n]
